# Initial kernel scaffold; baseline (speedup 1.0000x reference)
import jax
import jax.numpy as jnp
from jax import lax
from jax.experimental import pallas as pl
from jax.experimental.pallas import tpu as pltpu
from jax.experimental.pallas import tpu_sc as plsc

N_NODES = 10000
N_EDGES = 320000
D_FEAT = 128
OUT_DIM = 128
BN_EPS = 1e-3
INV_STD = 1.0 / (1.0 + BN_EPS) ** 0.5

NUM_CORES = 2
NUM_SUBCORES = 16
NUM_TILES = NUM_CORES * NUM_SUBCORES

E_PER_TILE = N_EDGES // NUM_TILES
CHUNK = 80
N_CHUNKS = E_PER_TILE // CHUNK
ROWS_PER_TILE = N_NODES // NUM_SUBCORES


def _seg_sum_body(dst_hbm, src_hbm, x_hbm, zeros_hbm, out_hbm,
                  agg_sh, dst_v, src_v, rows_v, gsem):
    c = lax.axis_index("c")
    s = lax.axis_index("s")
    w = c * NUM_SUBCORES + s
    edge_base = w * E_PER_TILE
    r0 = s * ROWS_PER_TILE

    pltpu.sync_copy(zeros_hbm.at[pl.ds(r0, ROWS_PER_TILE)],
                    agg_sh.at[pl.ds(r0, ROWS_PER_TILE)])
    plsc.subcore_barrier()

    def body(j, carry):
        b = edge_base + j * CHUNK
        pltpu.sync_copy(dst_hbm.at[pl.ds(b, CHUNK)], dst_v)
        pltpu.sync_copy(src_hbm.at[pl.ds(b, CHUNK)], src_v)
        pltpu.async_copy(x_hbm.at[src_v], rows_v, gsem).wait()
        pltpu.sync_copy(rows_v, agg_sh.at[dst_v], add=True)
        return carry

    lax.fori_loop(0, N_CHUNKS, body, 0)

    plsc.subcore_barrier()
    pltpu.sync_copy(agg_sh.at[pl.ds(r0, ROWS_PER_TILE)],
                    out_hbm.at[pl.ds(c * N_NODES + r0, ROWS_PER_TILE)])


def _segment_sum_sc(dst, src, x):
    zeros = jnp.zeros((N_NODES, D_FEAT), jnp.float32)
    mesh = plsc.VectorSubcoreMesh(core_axis_name="c", subcore_axis_name="s")
    f = pl.kernel(
        _seg_sum_body,
        out_type=jax.ShapeDtypeStruct((NUM_CORES * N_NODES, D_FEAT),
                                      jnp.float32),
        mesh=mesh,
        scratch_types=[
            pltpu.VMEM_SHARED((N_NODES, D_FEAT), jnp.float32),
            pltpu.VMEM((CHUNK,), jnp.int32),
            pltpu.VMEM((CHUNK,), jnp.int32),
            pltpu.VMEM((CHUNK, D_FEAT), jnp.float32),
            pltpu.SemaphoreType.DMA,
        ],
    )
    return f(dst, src, x, zeros)


def _mlp_body(p_ref, w1_ref, w2_ref, g_ref, b_ref, o_ref):
    h = p_ref[0] + p_ref[1]
    h1 = jnp.dot(h, w1_ref[...], preferred_element_type=jnp.float32)
    h1 = g_ref[...] * (h1 * INV_STD) + b_ref[...]
    h1 = jnp.maximum(h1, 0.0)
    o_ref[...] = jnp.dot(h1, w2_ref[...], preferred_element_type=jnp.float32)


def _mlp_tc(partials, W1, W2, gamma, beta):
    R = 1000
    grid = (N_NODES // R,)
    p3 = partials.reshape(NUM_CORES, N_NODES, D_FEAT)
    return pl.pallas_call(
        _mlp_body,
        grid=grid,
        in_specs=[
            pl.BlockSpec((NUM_CORES, R, D_FEAT), lambda i: (0, i, 0)),
            pl.BlockSpec((D_FEAT, OUT_DIM), lambda i: (0, 0)),
            pl.BlockSpec((OUT_DIM, OUT_DIM), lambda i: (0, 0)),
            pl.BlockSpec((1, OUT_DIM), lambda i: (0, 0)),
            pl.BlockSpec((1, OUT_DIM), lambda i: (0, 0)),
        ],
        out_specs=pl.BlockSpec((R, OUT_DIM), lambda i: (i, 0)),
        out_shape=jax.ShapeDtypeStruct((N_NODES, OUT_DIM), jnp.float32),
    )(p3, W1, W2, gamma.reshape(1, OUT_DIM), beta.reshape(1, OUT_DIM))


def kernel(x, edge_index, W1, W2, gamma, beta):
    dst = edge_index[0]
    src = edge_index[1]
    partials = _segment_sum_sc(dst, src, x)
    return _mlp_tc(partials, W1, W2, gamma, beta)


# speedup vs baseline: 5.4592x; 5.4592x over previous
import jax
import jax.numpy as jnp
from jax import lax
from jax.experimental import pallas as pl
from jax.experimental.pallas import tpu as pltpu
from jax.experimental.pallas import tpu_sc as plsc

N_NODES = 10000
N_EDGES = 320000
D_FEAT = 128
OUT_DIM = 128
BN_EPS = 1e-3
INV_STD = 1.0 / (1.0 + BN_EPS) ** 0.5

NUM_CORES = 2
NUM_SUBCORES = 16
NUM_TILES = NUM_CORES * NUM_SUBCORES

E_PER_TILE = N_EDGES // NUM_TILES
CHUNK = 80
N_CHUNKS = E_PER_TILE // CHUNK
N_PAD = 10240
ROWS_PER_TILE = N_PAD // NUM_SUBCORES


def _seg_sum_body(dst_hbm, src_hbm, x_hbm, zeros_hbm, out_hbm,
                  agg_sh, dst_v, src_v, rows_v, gsem):
    c = lax.axis_index("c")
    s = lax.axis_index("s")
    w = c * NUM_SUBCORES + s
    edge_base = w * E_PER_TILE
    r0 = s * ROWS_PER_TILE

    pltpu.sync_copy(zeros_hbm.at[pl.ds(r0, ROWS_PER_TILE)],
                    agg_sh.at[pl.ds(r0, ROWS_PER_TILE)])
    plsc.subcore_barrier()

    def body(j, carry):
        b = edge_base + j * CHUNK
        pltpu.sync_copy(dst_hbm.at[pl.ds(b, CHUNK)], dst_v)
        pltpu.sync_copy(src_hbm.at[pl.ds(b, CHUNK)], src_v)
        pltpu.async_copy(x_hbm.at[src_v], rows_v, gsem).wait()
        pltpu.sync_copy(rows_v, agg_sh.at[dst_v], add=True)
        return carry

    lax.fori_loop(0, N_CHUNKS, body, 0)

    plsc.subcore_barrier()
    pltpu.sync_copy(agg_sh.at[pl.ds(r0, ROWS_PER_TILE)],
                    out_hbm.at[pl.ds(c * N_PAD + r0, ROWS_PER_TILE)])


def _segment_sum_sc(dst, src, x):
    zeros = jnp.zeros((N_PAD, D_FEAT), jnp.float32)
    mesh = plsc.VectorSubcoreMesh(core_axis_name="c", subcore_axis_name="s")
    f = pl.kernel(
        _seg_sum_body,
        out_type=jax.ShapeDtypeStruct((NUM_CORES * N_PAD, D_FEAT),
                                      jnp.float32),
        mesh=mesh,
        scratch_types=[
            pltpu.VMEM_SHARED((N_PAD, D_FEAT), jnp.float32),
            pltpu.VMEM((CHUNK,), jnp.int32),
            pltpu.VMEM((CHUNK,), jnp.int32),
            pltpu.VMEM((CHUNK, D_FEAT), jnp.float32),
            pltpu.SemaphoreType.DMA,
        ],
    )
    return f(dst, src, x, zeros)


def _mlp_body(p_ref, w1_ref, w2_ref, g_ref, b_ref, o_ref):
    h = p_ref[0] + p_ref[1]
    h1 = jnp.dot(h, w1_ref[...], preferred_element_type=jnp.float32)
    h1 = g_ref[...] * (h1 * INV_STD) + b_ref[...]
    h1 = jnp.maximum(h1, 0.0)
    o_ref[...] = jnp.dot(h1, w2_ref[...], preferred_element_type=jnp.float32)


def _mlp_tc(partials, W1, W2, gamma, beta):
    R = 1000
    grid = (N_NODES // R,)
    p3 = partials.reshape(NUM_CORES, N_PAD, D_FEAT)
    return pl.pallas_call(
        _mlp_body,
        grid=grid,
        in_specs=[
            pl.BlockSpec((NUM_CORES, R, D_FEAT), lambda i: (0, i, 0)),
            pl.BlockSpec((D_FEAT, OUT_DIM), lambda i: (0, 0)),
            pl.BlockSpec((OUT_DIM, OUT_DIM), lambda i: (0, 0)),
            pl.BlockSpec((1, OUT_DIM), lambda i: (0, 0)),
            pl.BlockSpec((1, OUT_DIM), lambda i: (0, 0)),
        ],
        out_specs=pl.BlockSpec((R, OUT_DIM), lambda i: (i, 0)),
        out_shape=jax.ShapeDtypeStruct((N_NODES, OUT_DIM), jnp.float32),
    )(p3, W1, W2, gamma.reshape(1, OUT_DIM), beta.reshape(1, OUT_DIM))


def kernel(x, edge_index, W1, W2, gamma, beta):
    dst = edge_index[0]
    src = edge_index[1]
    partials = _segment_sum_sc(dst, src, x)
    return _mlp_tc(partials, W1, W2, gamma, beta)


# speedup vs baseline: 9.9116x; 1.8156x over previous
import jax
import jax.numpy as jnp
from jax import lax
from jax.experimental import pallas as pl
from jax.experimental.pallas import tpu as pltpu
from jax.experimental.pallas import tpu_sc as plsc

N_NODES = 10000
N_EDGES = 320000
D_FEAT = 128
OUT_DIM = 128
BN_EPS = 1e-3
INV_STD = 1.0 / (1.0 + BN_EPS) ** 0.5

NUM_CORES = 2
NUM_SUBCORES = 16
NUM_TILES = NUM_CORES * NUM_SUBCORES

E_PER_TILE = N_EDGES // NUM_TILES
CHUNK = 40
N_CHUNKS = E_PER_TILE // CHUNK
N_PAD = 10240
ROWS_PER_TILE = N_PAD // NUM_SUBCORES
RBUF = 3
IBUF = 4


def _seg_sum_body(dst_hbm, src_hbm, x_hbm, zeros_hbm, out_hbm,
                  agg_sh, dsti, srci, rows_v, isem, gsem, ssem):
    c = lax.axis_index("c")
    s = lax.axis_index("s")
    w = c * NUM_SUBCORES + s
    base = w * E_PER_TILE
    r0 = s * ROWS_PER_TILE

    pltpu.sync_copy(zeros_hbm.at[pl.ds(r0, ROWS_PER_TILE)],
                    agg_sh.at[pl.ds(r0, ROWS_PER_TILE)])
    plsc.subcore_barrier()

    for k in range(RBUF):
        pltpu.async_copy(dst_hbm.at[pl.ds(base + k * CHUNK, CHUNK)],
                         dsti.at[k], isem)
        pltpu.async_copy(src_hbm.at[pl.ds(base + k * CHUNK, CHUNK)],
                         srci.at[k], isem)
    for k in range(RBUF - 1):
        pltpu.make_async_copy(dst_hbm.at[pl.ds(0, CHUNK)],
                              dsti.at[k], isem).wait()
        pltpu.make_async_copy(dst_hbm.at[pl.ds(0, CHUNK)],
                              srci.at[k], isem).wait()
        pltpu.async_copy(x_hbm.at[srci.at[k]], rows_v.at[k], gsem)

    def body(j, carry):
        b = lax.rem(j, RBUF)
        i = lax.rem(j, IBUF)
        pltpu.make_async_copy(x_hbm.at[pl.ds(0, CHUNK)],
                              rows_v.at[b], gsem).wait()
        pltpu.async_copy(rows_v.at[b], agg_sh.at[dsti.at[i]], ssem, add=True)

        @pl.when(j >= 1)
        def _drain():
            pltpu.make_async_copy(x_hbm.at[pl.ds(0, CHUNK)],
                                  rows_v.at[b], ssem).wait()

        @pl.when(j + IBUF - 1 < N_CHUNKS)
        def _fire_idx():
            i3 = lax.rem(j + IBUF - 1, IBUF)
            b3 = base + (j + IBUF - 1) * CHUNK
            pltpu.async_copy(dst_hbm.at[pl.ds(b3, CHUNK)], dsti.at[i3], isem)
            pltpu.async_copy(src_hbm.at[pl.ds(b3, CHUNK)], srci.at[i3], isem)

        @pl.when(j + RBUF - 1 < N_CHUNKS)
        def _fire_gather():
            i2 = lax.rem(j + RBUF - 1, IBUF)
            b2 = lax.rem(j + RBUF - 1, RBUF)
            pltpu.make_async_copy(dst_hbm.at[pl.ds(0, CHUNK)],
                                  dsti.at[i2], isem).wait()
            pltpu.make_async_copy(dst_hbm.at[pl.ds(0, CHUNK)],
                                  srci.at[i2], isem).wait()
            pltpu.async_copy(x_hbm.at[srci.at[i2]], rows_v.at[b2], gsem)

        return carry

    lax.fori_loop(0, N_CHUNKS, body, 0)
    pltpu.make_async_copy(x_hbm.at[pl.ds(0, CHUNK)],
                          rows_v.at[0], ssem).wait()

    plsc.subcore_barrier()
    pltpu.sync_copy(agg_sh.at[pl.ds(r0, ROWS_PER_TILE)],
                    out_hbm.at[c, pl.ds(r0, ROWS_PER_TILE)])


def _segment_sum_sc(dst, src, x):
    zeros = jnp.zeros((N_PAD, D_FEAT), jnp.float32)
    mesh = plsc.VectorSubcoreMesh(core_axis_name="c", subcore_axis_name="s")
    f = pl.kernel(
        _seg_sum_body,
        out_type=jax.ShapeDtypeStruct((NUM_CORES, N_PAD, D_FEAT),
                                      jnp.float32),
        mesh=mesh,
        scratch_types=[
            pltpu.VMEM_SHARED((N_PAD, D_FEAT), jnp.float32),
            pltpu.VMEM((IBUF, CHUNK), jnp.int32),
            pltpu.VMEM((IBUF, CHUNK), jnp.int32),
            pltpu.VMEM((RBUF, CHUNK, D_FEAT), jnp.float32),
            pltpu.SemaphoreType.DMA,
            pltpu.SemaphoreType.DMA,
            pltpu.SemaphoreType.DMA,
        ],
    )
    return f(dst, src, x, zeros)


def _mlp_body(p_ref, w1_ref, w2_ref, g_ref, b_ref, o_ref):
    h = p_ref[0] + p_ref[1]
    h1 = jnp.dot(h, w1_ref[...], preferred_element_type=jnp.float32)
    h1 = g_ref[...] * (h1 * INV_STD) + b_ref[...]
    h1 = jnp.maximum(h1, 0.0)
    o_ref[...] = jnp.dot(h1, w2_ref[...], preferred_element_type=jnp.float32)


def _mlp_tc(partials, W1, W2, gamma, beta):
    R = 1000
    grid = (N_NODES // R,)
    return pl.pallas_call(
        _mlp_body,
        grid=grid,
        in_specs=[
            pl.BlockSpec((NUM_CORES, R, D_FEAT), lambda i: (0, i, 0)),
            pl.BlockSpec((D_FEAT, OUT_DIM), lambda i: (0, 0)),
            pl.BlockSpec((OUT_DIM, OUT_DIM), lambda i: (0, 0)),
            pl.BlockSpec((1, OUT_DIM), lambda i: (0, 0)),
            pl.BlockSpec((1, OUT_DIM), lambda i: (0, 0)),
        ],
        out_specs=pl.BlockSpec((R, OUT_DIM), lambda i: (i, 0)),
        out_shape=jax.ShapeDtypeStruct((N_NODES, OUT_DIM), jnp.float32),
    )(partials, W1, W2, gamma.reshape(1, OUT_DIM), beta.reshape(1, OUT_DIM))


def kernel(x, edge_index, W1, W2, gamma, beta):
    dst = edge_index[0]
    src = edge_index[1]
    partials = _segment_sum_sc(dst, src, x)
    return _mlp_tc(partials, W1, W2, gamma, beta)


# speedup vs baseline: 12.4252x; 1.2536x over previous
import jax
import jax.numpy as jnp
from jax import lax
from jax.experimental import pallas as pl
from jax.experimental.pallas import tpu as pltpu
from jax.experimental.pallas import tpu_sc as plsc

N_NODES = 10000
N_EDGES = 320000
D_FEAT = 128
OUT_DIM = 128
BN_EPS = 1e-3
INV_STD = 1.0 / (1.0 + BN_EPS) ** 0.5

NUM_CORES = 2
NUM_SUBCORES = 16
NUM_TILES = NUM_CORES * NUM_SUBCORES

E_PER_TILE = N_EDGES // NUM_TILES
CHUNK = 40
N_CHUNKS = E_PER_TILE // CHUNK
N_PAD = 10240
ROWS_PER_TILE = N_PAD // NUM_SUBCORES
RBUF = 3
IBUF = 4


def _seg_sum_body(dst_hbm, src_hbm, x_hbm, out_hbm,
                  agg_sh, dsti, srci, rows_v, isem, gsem, ssem):
    c = lax.axis_index("c")
    s = lax.axis_index("s")
    w = c * NUM_SUBCORES + s
    base = w * E_PER_TILE
    r0 = s * ROWS_PER_TILE

    zv = jnp.zeros((16,), jnp.float32)

    def zrow(i, carry):
        rows_v[0, lax.div(i, 8), pl.ds(lax.rem(i, 8) * 16, 16)] = zv
        return carry

    lax.fori_loop(0, CHUNK * 8, zrow, 0)
    for k in range(ROWS_PER_TILE // CHUNK):
        pltpu.sync_copy(rows_v.at[0],
                        agg_sh.at[pl.ds(r0 + k * CHUNK, CHUNK)])
    plsc.subcore_barrier()

    for k in range(RBUF):
        pltpu.async_copy(dst_hbm.at[pl.ds(base + k * CHUNK, CHUNK)],
                         dsti.at[k], isem)
        pltpu.async_copy(src_hbm.at[pl.ds(base + k * CHUNK, CHUNK)],
                         srci.at[k], isem)
    for k in range(RBUF - 1):
        pltpu.make_async_copy(dst_hbm.at[pl.ds(0, CHUNK)],
                              dsti.at[k], isem).wait()
        pltpu.make_async_copy(dst_hbm.at[pl.ds(0, CHUNK)],
                              srci.at[k], isem).wait()
        pltpu.async_copy(x_hbm.at[srci.at[k]], rows_v.at[k], gsem)

    def body(j, carry):
        b = lax.rem(j, RBUF)
        i = lax.rem(j, IBUF)
        pltpu.make_async_copy(x_hbm.at[pl.ds(0, CHUNK)],
                              rows_v.at[b], gsem).wait()
        pltpu.async_copy(rows_v.at[b], agg_sh.at[dsti.at[i]], ssem, add=True)

        @pl.when(j >= 1)
        def _drain():
            pltpu.make_async_copy(x_hbm.at[pl.ds(0, CHUNK)],
                                  rows_v.at[b], ssem).wait()

        @pl.when(j + IBUF - 1 < N_CHUNKS)
        def _fire_idx():
            i3 = lax.rem(j + IBUF - 1, IBUF)
            b3 = base + (j + IBUF - 1) * CHUNK
            pltpu.async_copy(dst_hbm.at[pl.ds(b3, CHUNK)], dsti.at[i3], isem)
            pltpu.async_copy(src_hbm.at[pl.ds(b3, CHUNK)], srci.at[i3], isem)

        @pl.when(j + RBUF - 1 < N_CHUNKS)
        def _fire_gather():
            i2 = lax.rem(j + RBUF - 1, IBUF)
            b2 = lax.rem(j + RBUF - 1, RBUF)
            pltpu.make_async_copy(dst_hbm.at[pl.ds(0, CHUNK)],
                                  dsti.at[i2], isem).wait()
            pltpu.make_async_copy(dst_hbm.at[pl.ds(0, CHUNK)],
                                  srci.at[i2], isem).wait()
            pltpu.async_copy(x_hbm.at[srci.at[i2]], rows_v.at[b2], gsem)

        return carry

    lax.fori_loop(0, N_CHUNKS, body, 0)
    pltpu.make_async_copy(x_hbm.at[pl.ds(0, CHUNK)],
                          rows_v.at[0], ssem).wait()

    plsc.subcore_barrier()
    pltpu.sync_copy(agg_sh.at[pl.ds(r0, ROWS_PER_TILE)],
                    out_hbm.at[c, pl.ds(r0, ROWS_PER_TILE)])


def _segment_sum_sc(dst, src, x):
    mesh = plsc.VectorSubcoreMesh(core_axis_name="c", subcore_axis_name="s")
    f = pl.kernel(
        _seg_sum_body,
        out_type=jax.ShapeDtypeStruct((NUM_CORES, N_PAD, D_FEAT),
                                      jnp.float32),
        mesh=mesh,
        scratch_types=[
            pltpu.VMEM_SHARED((N_PAD, D_FEAT), jnp.float32),
            pltpu.VMEM((IBUF, CHUNK), jnp.int32),
            pltpu.VMEM((IBUF, CHUNK), jnp.int32),
            pltpu.VMEM((RBUF, CHUNK, D_FEAT), jnp.float32),
            pltpu.SemaphoreType.DMA,
            pltpu.SemaphoreType.DMA,
            pltpu.SemaphoreType.DMA,
        ],
    )
    return f(dst, src, x)


def _mlp_body(p_ref, w1_ref, w2_ref, g_ref, b_ref, o_ref):
    h = p_ref[0] + p_ref[1]
    h1 = jnp.dot(h, w1_ref[...], preferred_element_type=jnp.float32)
    h1 = g_ref[...] * (h1 * INV_STD) + b_ref[...]
    h1 = jnp.maximum(h1, 0.0)
    o_ref[...] = jnp.dot(h1, w2_ref[...], preferred_element_type=jnp.float32)


def _mlp_tc(partials, W1, W2, gamma, beta):
    R = 1000
    grid = (N_NODES // R,)
    return pl.pallas_call(
        _mlp_body,
        grid=grid,
        in_specs=[
            pl.BlockSpec((NUM_CORES, R, D_FEAT), lambda i: (0, i, 0)),
            pl.BlockSpec((D_FEAT, OUT_DIM), lambda i: (0, 0)),
            pl.BlockSpec((OUT_DIM, OUT_DIM), lambda i: (0, 0)),
            pl.BlockSpec((1, OUT_DIM), lambda i: (0, 0)),
            pl.BlockSpec((1, OUT_DIM), lambda i: (0, 0)),
        ],
        out_specs=pl.BlockSpec((R, OUT_DIM), lambda i: (i, 0)),
        out_shape=jax.ShapeDtypeStruct((N_NODES, OUT_DIM), jnp.float32),
    )(partials, W1, W2, gamma.reshape(1, OUT_DIM), beta.reshape(1, OUT_DIM))


def kernel(x, edge_index, W1, W2, gamma, beta):
    dst = edge_index[0]
    src = edge_index[1]
    partials = _segment_sum_sc(dst, src, x)
    return _mlp_tc(partials, W1, W2, gamma, beta)
